# Initial kernel scaffold; baseline (speedup 1.0000x reference)
import jax
import jax.numpy as jnp
from jax.experimental import pallas as pl


def kernel(x, edge_index, batch, nc0_Wrel, nc0_brel, nc0_Wroot, nc1_Wrel, nc1_brel, nc1_Wroot, lgl_W0, lgl_b0, lgl_W1, lgl_b1, temp, theta, g0_Wrel, g0_brel, g0_Wroot, g1_Wrel, g1_brel, g1_Wroot, cls_W0, cls_b0, cls_W1, cls_b1):
    raise NotImplementedError("write your pallas kernel here")


# speedup vs baseline: 11.2575x; 11.2575x over previous
import functools

import jax
import jax.numpy as jnp
import numpy as np
from jax import lax
from jax.experimental import pallas as pl
from jax.experimental.pallas import tpu as pltpu
from jax.experimental.pallas import tpu_sc as plsc

N, E, B, D = 10000, 320000, 512, 128
EPS = float(np.finfo(np.float32).eps)

NC, NS = 2, 16
NW = NC * NS
CH = 128
NCHUNKS = E // CH
CH_PER_TILE = NCHUNKS // NW
CH_REM = NCHUNKS - CH_PER_TILE * NW
RCH = 80
NRCH = N // RCH
RCH_PER_TILE = NRCH // NS
RCH_REM = NRCH - RCH_PER_TILE * NS


def _seg_sum_body(x_hbm, src_hbm, dst_hbm, out_hbm, src_v, dst_v, rows_v, acc_sh, sem):
    cid = lax.axis_index("c")
    sid = lax.axis_index("s")
    wid = sid * NC + cid

    def _zero_row(r, carry):
        for k in range(D // 16):
            rows_v[r, pl.ds(k * 16, 16)] = jnp.zeros((16,), jnp.float32)
        return carry

    lax.fori_loop(0, RCH, _zero_row, 0)
    nrch = RCH_PER_TILE + jnp.where(sid < RCH_REM, 1, 0)
    rch_base = RCH_PER_TILE * sid + jnp.minimum(sid, RCH_REM)

    def _init_chunk(j, carry):
        pltpu.sync_copy(rows_v.at[pl.ds(0, RCH), :],
                        acc_sh.at[pl.ds((rch_base + j) * RCH, RCH), :])
        return carry

    lax.fori_loop(0, nrch, _init_chunk, 0)
    plsc.subcore_barrier()

    nch = CH_PER_TILE + jnp.where(wid < CH_REM, 1, 0)
    ch_base = CH_PER_TILE * wid + jnp.minimum(wid, CH_REM)

    def _step(c, carry):
        e0 = (ch_base + c) * CH
        pltpu.sync_copy(src_hbm.at[pl.ds(e0, CH)], src_v)
        pltpu.sync_copy(dst_hbm.at[pl.ds(e0, CH)], dst_v)
        pltpu.async_copy(x_hbm.at[src_v], rows_v, sem).wait()
        pltpu.sync_copy(rows_v, acc_sh.at[dst_v], add=True)
        return carry

    lax.fori_loop(0, nch, _step, 0)
    plsc.subcore_barrier()

    def _out_chunk(j, carry):
        r0 = (rch_base + j) * RCH
        pltpu.sync_copy(acc_sh.at[pl.ds(r0, RCH), :], rows_v.at[pl.ds(0, RCH), :])
        pltpu.sync_copy(rows_v.at[pl.ds(0, RCH), :],
                        out_hbm.at[cid, pl.ds(r0, RCH), :])
        return carry

    lax.fori_loop(0, nrch, _out_chunk, 0)


@functools.cache
def _get_seg_sum():
    return pl.kernel(
        _seg_sum_body,
        out_type=jax.ShapeDtypeStruct((NC, N, D), jnp.float32),
        mesh=plsc.VectorSubcoreMesh(core_axis_name="c", subcore_axis_name="s",
                                    num_cores=NC, num_subcores=NS),
        scratch_types=[
            pltpu.VMEM((CH,), jnp.int32),
            pltpu.VMEM((CH,), jnp.int32),
            pltpu.VMEM((CH, D), jnp.float32),
            pltpu.VMEM_SHARED((N, D), jnp.float32),
            pltpu.SemaphoreType.DMA,
        ],
    )


def _conv_post_body(p_ref, x_ref, wrelT_ref, wrootT_ref, brel_ref, o_ref):
    agg = p_ref[0] + p_ref[1]
    o_ref[...] = jnp.maximum(
        jnp.dot(agg, wrelT_ref[...], preferred_element_type=jnp.float32)
        + jnp.dot(x_ref[...], wrootT_ref[...], preferred_element_type=jnp.float32)
        + brel_ref[...],
        0.0,
    )


_NB = 400


def _conv_post(p, x, wrelT, wrootT, brel2d):
    return pl.pallas_call(
        _conv_post_body,
        grid=(N // _NB,),
        in_specs=[
            pl.BlockSpec((NC, _NB, D), lambda i: (0, i, 0)),
            pl.BlockSpec((_NB, D), lambda i: (i, 0)),
            pl.BlockSpec((D, D), lambda i: (0, 0)),
            pl.BlockSpec((D, D), lambda i: (0, 0)),
            pl.BlockSpec((1, D), lambda i: (0, 0)),
        ],
        out_specs=pl.BlockSpec((_NB, D), lambda i: (i, 0)),
        out_shape=jax.ShapeDtypeStruct((N, D), jnp.float32),
    )(p, x, wrelT, wrootT, brel2d)


_PCH = 2000


def _tail_body(h_ref, b_ref, lw0T_ref, lb0_ref, lw1T_ref, lb1_ref, tt_ref, th_ref,
               g0rT_ref, g0b_ref, g0oT_ref, g1rT_ref, g1b_ref, g1oT_ref,
               cw0T_ref, cb0_ref, cw1T_ref, cb1_ref, o_ref):
    f32 = jnp.float32
    ids = lax.broadcasted_iota(jnp.int32, (B, 1), 0)
    ssum = jnp.zeros((B, D), f32)
    cnt = jnp.zeros((B, 1), f32)
    for r in range(N // _PCH):
        row = b_ref[r, :]
        m = (row[None, :] == ids).astype(f32)
        ssum = ssum + jnp.dot(m, h_ref[r * _PCH:(r + 1) * _PCH, :],
                              preferred_element_type=f32, precision=lax.Precision.HIGHEST)
        cnt = cnt + jnp.sum(m, axis=1, keepdims=True)
    feat = ssum / jnp.maximum(cnt, 1.0)

    o1 = jnp.maximum(jnp.dot(feat, lw0T_ref[...], preferred_element_type=f32)
                     + lb0_ref[...], 0.0)
    o2 = jnp.maximum(jnp.dot(o1, lw1T_ref[...], preferred_element_type=f32)
                     + lb1_ref[...], 0.0)

    G = lax.dot_general(o2, o2, (((1,), (1,)), ((), ())),
                        preferred_element_type=f32, precision=lax.Precision.HIGHEST)
    eye = (lax.broadcasted_iota(jnp.int32, (B, B), 0)
           == lax.broadcasted_iota(jnp.int32, (B, B), 1)).astype(f32)
    ncol = jnp.sum(G * eye, axis=1, keepdims=True)
    nrow = jnp.sum(G * eye, axis=0, keepdims=True)
    d2 = jnp.maximum(ncol + nrow - 2.0 * G, 0.0)
    msk = (d2 != 0.0).astype(f32)
    dist = -jnp.sqrt(d2 + EPS) * msk
    prob = tt_ref[0, 0] * dist + th_ref[0, 0]
    adjm = jax.nn.sigmoid(prob + eye)

    agg0 = lax.dot_general(adjm, feat, (((0,), (0,)), ((), ())),
                           preferred_element_type=f32, precision=lax.Precision.HIGHEST)
    g = jnp.maximum(jnp.dot(agg0, g0rT_ref[...], preferred_element_type=f32)
                    + g0b_ref[...]
                    + jnp.dot(feat, g0oT_ref[...], preferred_element_type=f32), 0.0)
    rowvec = jnp.dot(jnp.sum(g, axis=0, keepdims=True), g1rT_ref[...],
                     preferred_element_type=f32) + g1b_ref[...]
    g2 = jnp.maximum(jnp.dot(g, g1oT_ref[...], preferred_element_type=f32)
                     + rowvec, 0.0)
    c1 = jnp.maximum(jnp.dot(g2, cw0T_ref[...], preferred_element_type=f32)
                     + cb0_ref[...], 0.0)
    o_ref[...] = jnp.dot(c1, cw1T_ref[...], preferred_element_type=f32) + cb1_ref[0, 0]


def _tail(h, batch2d, *ws):
    return pl.pallas_call(
        _tail_body,
        out_shape=jax.ShapeDtypeStruct((B, D), jnp.float32),
    )(h, batch2d, *ws)


def kernel(x, edge_index, batch, nc0_Wrel, nc0_brel, nc0_Wroot, nc1_Wrel, nc1_brel,
           nc1_Wroot, lgl_W0, lgl_b0, lgl_W1, lgl_b1, temp, theta,
           g0_Wrel, g0_brel, g0_Wroot, g1_Wrel, g1_brel, g1_Wroot,
           cls_W0, cls_b0, cls_W1, cls_b1):
    src = edge_index[0]
    dst = edge_index[1]
    batch2d = batch.reshape(N // _PCH, _PCH)

    seg_sum = _get_seg_sum()
    p0 = seg_sum(x, src, dst)
    h1 = _conv_post(p0, x, nc0_Wrel.T, nc0_Wroot.T, nc0_brel[None, :])
    p1 = seg_sum(h1, src, dst)
    h2 = _conv_post(p1, h1, nc1_Wrel.T, nc1_Wroot.T, nc1_brel[None, :])

    cw1T = jnp.zeros((64, D), jnp.float32).at[:, :1].set(cls_W1.T)
    out = _tail(
        h2, batch2d,
        lgl_W0.T, lgl_b0[None, :], lgl_W1.T, lgl_b1[None, :],
        temp.reshape(1, 1), theta.reshape(1, 1),
        g0_Wrel.T, g0_brel[None, :], g0_Wroot.T,
        g1_Wrel.T, g1_brel[None, :], g1_Wroot.T,
        cls_W0.T, cls_b0[None, :], cw1T, cls_b1.reshape(1, 1),
    )
    return out[:, :1]


# speedup vs baseline: 19.6986x; 1.7498x over previous
import functools

import jax
import jax.numpy as jnp
import numpy as np
from jax import lax
from jax.experimental import pallas as pl
from jax.experimental.pallas import tpu as pltpu
from jax.experimental.pallas import tpu_sc as plsc

N, E, B, D = 10000, 320000, 512, 128
EPS = float(np.finfo(np.float32).eps)

NC, NS = 2, 16
NW = NC * NS
CH = 128
NCHUNKS = E // CH
CHT = NCHUNKS // NW
CREM = NCHUNKS - CHT * NW
RCH = 80
NRCH = N // RCH
RCH_PER_TILE = NRCH // NS
RCH_REM = NRCH - RCH_PER_TILE * NS


def _seg_sum_body(x_hbm, src_hbm, dst_hbm, out_hbm, src_v0, src_v1, src_v2,
                  dst_v0, dst_v1, dst_v2, rows_v, acc_sh,
                  gs0, gs1, gs2, ss0, ss1, ss2):
    cid = lax.axis_index("c")
    sid = lax.axis_index("s")
    wid = sid * NC + cid

    def _zero_row(r, carry):
        for k in range(D // 16):
            rows_v[0, r, pl.ds(k * 16, 16)] = jnp.zeros((16,), jnp.float32)
        return carry

    lax.fori_loop(0, RCH, _zero_row, 0)
    nrch = RCH_PER_TILE + jnp.where(sid < RCH_REM, 1, 0)
    rch_base = RCH_PER_TILE * sid + jnp.minimum(sid, RCH_REM)

    def _init_chunk(j, carry):
        pltpu.sync_copy(rows_v.at[0, pl.ds(0, RCH), :],
                        acc_sh.at[pl.ds((rch_base + j) * RCH, RCH), :])
        return carry

    lax.fori_loop(0, nrch, _init_chunk, 0)
    plsc.subcore_barrier()

    ch0 = wid * CHT
    gsems = (gs0, gs1, gs2)
    ssems = (ss0, ss1, ss2)
    srcs = (src_v0, src_v1, src_v2)
    dsts = (dst_v0, dst_v1, dst_v2)
    for b in range(2):
        pltpu.sync_copy(src_hbm.at[pl.ds((ch0 + b) * CH, CH)], srcs[b])
        pltpu.sync_copy(dst_hbm.at[pl.ds((ch0 + b) * CH, CH)], dsts[b])
        pltpu.async_copy(x_hbm.at[srcs[b]], rows_v.at[b], gsems[b])

    def _outer(g, carry):
        for b0 in range(3):
            c = 3 * g + b0
            pltpu.make_async_copy(x_hbm.at[srcs[b0]], rows_v.at[b0],
                                  gsems[b0]).wait()
            pltpu.async_copy(rows_v.at[b0], acc_sh.at[dsts[b0]], ssems[b0],
                             add=True)
            b2 = (b0 + 2) % 3

            @pl.when(jnp.logical_and(c >= 1, c <= CHT - 3))
            def _():
                pltpu.make_async_copy(rows_v.at[b2], acc_sh.at[dsts[b2]],
                                      ssems[b2]).wait()

            @pl.when(c <= CHT - 3)
            def _():
                e0 = (ch0 + c + 2) * CH
                pltpu.sync_copy(src_hbm.at[pl.ds(e0, CH)], srcs[b2])
                pltpu.sync_copy(dst_hbm.at[pl.ds(e0, CH)], dsts[b2])
                pltpu.async_copy(x_hbm.at[srcs[b2]], rows_v.at[b2], gsems[b2])
        return carry

    lax.fori_loop(0, CHT // 3, _outer, 0)
    for b in range(3):
        pltpu.make_async_copy(rows_v.at[b], acc_sh.at[dsts[b]],
                              ssems[b]).wait()

    @pl.when(wid < CREM)
    def _rem():
        e0 = (CHT * NW + wid) * CH
        pltpu.sync_copy(src_hbm.at[pl.ds(e0, CH)], src_v0)
        pltpu.sync_copy(dst_hbm.at[pl.ds(e0, CH)], dst_v0)
        pltpu.async_copy(x_hbm.at[src_v0], rows_v.at[0], gs0).wait()
        pltpu.sync_copy(rows_v.at[0], acc_sh.at[dst_v0], add=True)

    plsc.subcore_barrier()

    def _out_chunk(j, carry):
        r0 = (rch_base + j) * RCH
        pltpu.sync_copy(acc_sh.at[pl.ds(r0, RCH), :], rows_v.at[0, pl.ds(0, RCH), :])
        pltpu.sync_copy(rows_v.at[0, pl.ds(0, RCH), :],
                        out_hbm.at[cid, pl.ds(r0, RCH), :])
        return carry

    lax.fori_loop(0, nrch, _out_chunk, 0)


@functools.cache
def _get_seg_sum():
    return pl.kernel(
        _seg_sum_body,
        out_type=jax.ShapeDtypeStruct((NC, N, D), jnp.float32),
        mesh=plsc.VectorSubcoreMesh(core_axis_name="c", subcore_axis_name="s",
                                    num_cores=NC, num_subcores=NS),
        scratch_types=[
            pltpu.VMEM((CH,), jnp.int32),
            pltpu.VMEM((CH,), jnp.int32),
            pltpu.VMEM((CH,), jnp.int32),
            pltpu.VMEM((CH,), jnp.int32),
            pltpu.VMEM((CH,), jnp.int32),
            pltpu.VMEM((CH,), jnp.int32),
            pltpu.VMEM((3, CH, D), jnp.float32),
            pltpu.VMEM_SHARED((N, D), jnp.float32),
            pltpu.SemaphoreType.DMA,
            pltpu.SemaphoreType.DMA,
            pltpu.SemaphoreType.DMA,
            pltpu.SemaphoreType.DMA,
            pltpu.SemaphoreType.DMA,
            pltpu.SemaphoreType.DMA,
        ],
    )


def _conv_post_body(p_ref, x_ref, wrelT_ref, wrootT_ref, brel_ref, o_ref):
    agg = p_ref[0] + p_ref[1]
    o_ref[...] = jnp.maximum(
        jnp.dot(agg, wrelT_ref[...], preferred_element_type=jnp.float32)
        + jnp.dot(x_ref[...], wrootT_ref[...], preferred_element_type=jnp.float32)
        + brel_ref[...],
        0.0,
    )


_NB = 400


def _conv_post(p, x, wrelT, wrootT, brel2d):
    return pl.pallas_call(
        _conv_post_body,
        grid=(N // _NB,),
        in_specs=[
            pl.BlockSpec((NC, _NB, D), lambda i: (0, i, 0)),
            pl.BlockSpec((_NB, D), lambda i: (i, 0)),
            pl.BlockSpec((D, D), lambda i: (0, 0)),
            pl.BlockSpec((D, D), lambda i: (0, 0)),
            pl.BlockSpec((1, D), lambda i: (0, 0)),
        ],
        out_specs=pl.BlockSpec((_NB, D), lambda i: (i, 0)),
        out_shape=jax.ShapeDtypeStruct((N, D), jnp.float32),
    )(p, x, wrelT, wrootT, brel2d)


_PCH = 2000


def _tail_body(h_ref, b_ref, lw0T_ref, lb0_ref, lw1T_ref, lb1_ref, tt_ref, th_ref,
               g0rT_ref, g0b_ref, g0oT_ref, g1rT_ref, g1b_ref, g1oT_ref,
               cw0T_ref, cb0_ref, cw1T_ref, cb1_ref, o_ref):
    f32 = jnp.float32
    ids = lax.broadcasted_iota(jnp.int32, (B, 1), 0)
    ssum = jnp.zeros((B, D), f32)
    cnt = jnp.zeros((B, 1), f32)
    for r in range(N // _PCH):
        row = b_ref[r, :]
        m = (row[None, :] == ids).astype(f32)
        ssum = ssum + jnp.dot(m, h_ref[r * _PCH:(r + 1) * _PCH, :],
                              preferred_element_type=f32, precision=lax.Precision.HIGHEST)
        cnt = cnt + jnp.sum(m, axis=1, keepdims=True)
    feat = ssum / jnp.maximum(cnt, 1.0)

    o1 = jnp.maximum(jnp.dot(feat, lw0T_ref[...], preferred_element_type=f32)
                     + lb0_ref[...], 0.0)
    o2 = jnp.maximum(jnp.dot(o1, lw1T_ref[...], preferred_element_type=f32)
                     + lb1_ref[...], 0.0)

    G = lax.dot_general(o2, o2, (((1,), (1,)), ((), ())),
                        preferred_element_type=f32, precision=lax.Precision.HIGHEST)
    eye = (lax.broadcasted_iota(jnp.int32, (B, B), 0)
           == lax.broadcasted_iota(jnp.int32, (B, B), 1)).astype(f32)
    ncol = jnp.sum(G * eye, axis=1, keepdims=True)
    nrow = jnp.sum(G * eye, axis=0, keepdims=True)
    d2 = jnp.maximum(ncol + nrow - 2.0 * G, 0.0)
    msk = (d2 != 0.0).astype(f32)
    dist = -jnp.sqrt(d2 + EPS) * msk
    prob = tt_ref[0, 0] * dist + th_ref[0, 0]
    adjm = jax.nn.sigmoid(prob + eye)

    agg0 = lax.dot_general(adjm, feat, (((0,), (0,)), ((), ())),
                           preferred_element_type=f32, precision=lax.Precision.HIGHEST)
    g = jnp.maximum(jnp.dot(agg0, g0rT_ref[...], preferred_element_type=f32)
                    + g0b_ref[...]
                    + jnp.dot(feat, g0oT_ref[...], preferred_element_type=f32), 0.0)
    rowvec = jnp.dot(jnp.sum(g, axis=0, keepdims=True), g1rT_ref[...],
                     preferred_element_type=f32) + g1b_ref[...]
    g2 = jnp.maximum(jnp.dot(g, g1oT_ref[...], preferred_element_type=f32)
                     + rowvec, 0.0)
    c1 = jnp.maximum(jnp.dot(g2, cw0T_ref[...], preferred_element_type=f32)
                     + cb0_ref[...], 0.0)
    o_ref[...] = jnp.dot(c1, cw1T_ref[...], preferred_element_type=f32) + cb1_ref[0, 0]


def _tail(h, batch2d, *ws):
    return pl.pallas_call(
        _tail_body,
        out_shape=jax.ShapeDtypeStruct((B, D), jnp.float32),
    )(h, batch2d, *ws)


def kernel(x, edge_index, batch, nc0_Wrel, nc0_brel, nc0_Wroot, nc1_Wrel, nc1_brel,
           nc1_Wroot, lgl_W0, lgl_b0, lgl_W1, lgl_b1, temp, theta,
           g0_Wrel, g0_brel, g0_Wroot, g1_Wrel, g1_brel, g1_Wroot,
           cls_W0, cls_b0, cls_W1, cls_b1):
    src = edge_index[0]
    dst = edge_index[1]
    batch2d = batch.reshape(N // _PCH, _PCH)

    seg_sum = _get_seg_sum()
    p0 = seg_sum(x, src, dst)
    h1 = _conv_post(p0, x, nc0_Wrel.T, nc0_Wroot.T, nc0_brel[None, :])
    p1 = seg_sum(h1, src, dst)
    h2 = _conv_post(p1, h1, nc1_Wrel.T, nc1_Wroot.T, nc1_brel[None, :])

    cw1T = jnp.zeros((64, D), jnp.float32).at[:, :1].set(cls_W1.T)
    out = _tail(
        h2, batch2d,
        lgl_W0.T, lgl_b0[None, :], lgl_W1.T, lgl_b1[None, :],
        temp.reshape(1, 1), theta.reshape(1, 1),
        g0_Wrel.T, g0_brel[None, :], g0_Wroot.T,
        g1_Wrel.T, g1_brel[None, :], g1_Wroot.T,
        cls_W0.T, cls_b0[None, :], cw1T, cls_b1.reshape(1, 1),
    )
    return out[:, :1]
